# Initial kernel scaffold; baseline (speedup 1.0000x reference)
import jax
import jax.numpy as jnp
from jax.experimental import pallas as pl


def kernel(flat_origin_curves, flat_new_curves, dict_points, dict_ref, dict_bool):
    raise NotImplementedError("write your pallas kernel here")


# speedup vs baseline: 1.4936x; 1.4936x over previous
import functools

import jax
import jax.numpy as jnp
from jax import lax
from jax.experimental import pallas as pl
from jax.experimental.pallas import tpu as pltpu
from jax.experimental.pallas import tpu_sc as plsc

_L = 16
_NC = 2
_NS = 16
_NW = _NC * _NS


def _make_sc_nn(n, k):
    qpw = n // _NW
    ng = qpw // _L
    mesh = plsc.VectorSubcoreMesh(core_axis_name="c", subcore_axis_name="s")

    @functools.partial(
        pl.kernel,
        out_type=[
            jax.ShapeDtypeStruct((n,), jnp.float32),
            jax.ShapeDtypeStruct((n,), jnp.float32),
        ],
        mesh=mesh,
        compiler_params=pltpu.CompilerParams(needs_layout_passes=False),
        scratch_types=[
            pltpu.VMEM((k,), jnp.float32),
            pltpu.VMEM((k,), jnp.float32),
            pltpu.VMEM((k,), jnp.float32),
            pltpu.VMEM((k,), jnp.float32),
            pltpu.VMEM((k,), jnp.float32),
            pltpu.VMEM((qpw,), jnp.float32),
            pltpu.VMEM((qpw,), jnp.float32),
            pltpu.VMEM((qpw,), jnp.float32),
            pltpu.VMEM((qpw,), jnp.float32),
            pltpu.VMEM((qpw,), jnp.float32),
            pltpu.VMEM((qpw,), jnp.float32),
        ],
    )
    def sc_nn(ox_h, oy_h, nx_h, ny_h, rx_h, ry_h, px_h, py_h, bf_h,
              d2_h, mk_h,
              rx_v, ry_v, px_v, py_v, bf_v,
              qx_v, qy_v, nx_v, ny_v, od_v, om_v):
        wid = lax.axis_index("s") * _NC + lax.axis_index("c")
        base = wid * qpw
        pltpu.sync_copy(rx_h, rx_v)
        pltpu.sync_copy(ry_h, ry_v)
        pltpu.sync_copy(px_h, px_v)
        pltpu.sync_copy(py_h, py_v)
        pltpu.sync_copy(bf_h, bf_v)
        pltpu.sync_copy(ox_h.at[pl.ds(base, qpw)], qx_v)
        pltpu.sync_copy(oy_h.at[pl.ds(base, qpw)], qy_v)
        pltpu.sync_copy(nx_h.at[pl.ds(base, qpw)], nx_v)
        pltpu.sync_copy(ny_h.at[pl.ds(base, qpw)], ny_v)

        qx = [qx_v[pl.ds(g * _L, _L)] for g in range(ng)]
        qy = [qy_v[pl.ds(g * _L, _L)] for g in range(ng)]
        inf = jnp.full((_L,), jnp.inf, jnp.float32)
        zero = jnp.zeros((_L,), jnp.int32)
        init = tuple([inf] * ng + [zero] * ng + [zero])

        def step(_, carry):
            st = list(carry)
            kv = st[2 * ng]
            rxb = plsc.load_gather(rx_v, [kv])
            ryb = plsc.load_gather(ry_v, [kv])
            for g in range(ng):
                dx = rxb - qx[g]
                dy = ryb - qy[g]
                d2 = dx * dx + dy * dy
                pred = d2 < st[g]
                st[g] = jnp.where(pred, d2, st[g])
                st[ng + g] = jnp.where(pred, kv, st[ng + g])
            st[2 * ng] = kv + 1
            return tuple(st)

        fin = lax.fori_loop(0, k, step, init, unroll=2)
        for g in range(ng):
            bid = fin[ng + g]
            pxg = plsc.load_gather(px_v, [bid])
            pyg = plsc.load_gather(py_v, [bid])
            bfg = plsc.load_gather(bf_v, [bid])
            ddx = nx_v[pl.ds(g * _L, _L)] - pxg
            ddy = ny_v[pl.ds(g * _L, _L)] - pyg
            od_v[pl.ds(g * _L, _L)] = ddx * ddx + ddy * ddy
            om_v[pl.ds(g * _L, _L)] = bfg
        pltpu.sync_copy(od_v, d2_h.at[pl.ds(base, qpw)])
        pltpu.sync_copy(om_v, mk_h.at[pl.ds(base, qpw)])

    return sc_nn


def _finish_body(d2_ref, mk_ref, out_ref):
    d = jnp.sqrt(d2_ref[...])
    m = mk_ref[...]
    out_ref[0, 0] = jnp.sum(d * m) / jnp.sum(m)


def _make_finish():
    return pl.pallas_call(
        _finish_body,
        out_shape=jax.ShapeDtypeStruct((1, 1), jnp.float32),
        out_specs=pl.BlockSpec(memory_space=pltpu.SMEM),
    )


def kernel(flat_origin_curves, flat_new_curves, dict_points, dict_ref, dict_bool):
    n = flat_origin_curves.shape[0]
    k = dict_ref.shape[0]
    ox = flat_origin_curves[:, 0]
    oy = flat_origin_curves[:, 1]
    nx = flat_new_curves[:, 0]
    ny = flat_new_curves[:, 1]
    rx = dict_ref[:, 0]
    ry = dict_ref[:, 1]
    px = dict_points[:, 0]
    py = dict_points[:, 1]
    bf = dict_bool.astype(jnp.float32)
    d2, mk = _make_sc_nn(n, k)(ox, oy, nx, ny, rx, ry, px, py, bf)
    loss = _make_finish()(d2.reshape(n // 128, 128), mk.reshape(n // 128, 128))
    return loss[0, 0]
